# Initial kernel scaffold; baseline (speedup 1.0000x reference)
import functools

import jax
import jax.numpy as jnp
from jax import lax
from jax.experimental import pallas as pl
from jax.experimental.pallas import tpu as pltpu
from jax.experimental.pallas import tpu_sc as plsc

NC = 2
NS = 16
NW = NC * NS
LANES = 128


def _mesh():
    return plsc.VectorSubcoreMesh(core_axis_name="c", subcore_axis_name="s")


def _make_deg_kernel(n_chunks, n_pad):
    rows_per_tile = n_pad // NS

    @functools.partial(
        pl.kernel,
        out_type=jax.ShapeDtypeStruct((NC, n_pad, 1), jnp.float32),
        mesh=_mesh(),
        scratch_types=[
            pltpu.VMEM((n_chunks, LANES), jnp.int32),
            pltpu.VMEM((LANES, 1), jnp.float32),
            pltpu.VMEM((rows_per_tile, 1), jnp.float32),
            pltpu.VMEM_SHARED((n_pad, 1), jnp.float32),
        ],
    )
    def deg_kernel(dst_hbm, ones_hbm, zeros_hbm, out_hbm, dstv, ones_v, zv, deg_s):
        cid = lax.axis_index("c")
        sid = lax.axis_index("s")
        wid = cid * NS + sid
        r0 = sid * rows_per_tile
        pltpu.sync_copy(dst_hbm.at[wid], dstv)
        pltpu.sync_copy(ones_hbm, ones_v)
        pltpu.sync_copy(zeros_hbm.at[pl.ds(r0, rows_per_tile)], zv)
        pltpu.sync_copy(zv, deg_s.at[pl.ds(r0, rows_per_tile)])
        plsc.subcore_barrier()

        def body(j, carry):
            pltpu.sync_copy(ones_v, deg_s.at[dstv.at[j]], add=True)
            return carry

        lax.fori_loop(0, n_chunks, body, 0)
        plsc.subcore_barrier()
        pltpu.sync_copy(deg_s.at[pl.ds(r0, rows_per_tile)], zv)
        pltpu.sync_copy(zv, out_hbm.at[cid, pl.ds(r0, rows_per_tile)])

    return deg_kernel


def _make_scatter_kernel(n_chunks, n_pad, d):
    rows_per_tile = n_pad // NS

    @functools.partial(
        pl.kernel,
        out_type=jax.ShapeDtypeStruct((NC, n_pad, d), jnp.float32),
        mesh=_mesh(),
        scratch_types=[
            pltpu.VMEM((n_chunks, LANES), jnp.int32),
            pltpu.VMEM((n_chunks, LANES), jnp.int32),
            pltpu.VMEM((LANES, d), jnp.float32),
            pltpu.VMEM((rows_per_tile, d), jnp.float32),
            pltpu.VMEM_SHARED((n_pad, d), jnp.float32),
        ],
    )
    def scatter_kernel(src_hbm, dst_hbm, y_hbm, zeros_hbm, out_hbm,
                       srcv, dstv, ybuf, zv, tmp_s):
        cid = lax.axis_index("c")
        sid = lax.axis_index("s")
        wid = cid * NS + sid
        r0 = sid * rows_per_tile
        pltpu.sync_copy(src_hbm.at[wid], srcv)
        pltpu.sync_copy(dst_hbm.at[wid], dstv)
        pltpu.sync_copy(zeros_hbm.at[pl.ds(r0, rows_per_tile)], zv)
        pltpu.sync_copy(zv, tmp_s.at[pl.ds(r0, rows_per_tile)])
        plsc.subcore_barrier()

        def body(j, carry):
            pltpu.sync_copy(y_hbm.at[srcv.at[j]], ybuf)
            pltpu.sync_copy(ybuf, tmp_s.at[dstv.at[j]], add=True)
            return carry

        lax.fori_loop(0, n_chunks, body, 0)
        plsc.subcore_barrier()
        pltpu.sync_copy(tmp_s.at[pl.ds(r0, rows_per_tile)], zv)
        pltpu.sync_copy(zv, out_hbm.at[cid, pl.ds(r0, rows_per_tile)])

    return scatter_kernel


def _y_dis_tc(x, w, d0, d1, block_rows):
    n = x.shape[0]
    grid = n // block_rows

    def body(x_ref, w_ref, d0_ref, d1_ref, y_ref, dis_ref):
        deg = d0_ref[...] + d1_ref[...] + 1.0
        dis = lax.rsqrt(deg)
        xw = jnp.dot(x_ref[...], w_ref[...], preferred_element_type=jnp.float32)
        y_ref[...] = xw * dis
        dis_ref[...] = dis

    return pl.pallas_call(
        body,
        grid=(grid,),
        in_specs=[
            pl.BlockSpec((block_rows, x.shape[1]), lambda i: (i, 0)),
            pl.BlockSpec((w.shape[0], w.shape[1]), lambda i: (0, 0)),
            pl.BlockSpec((block_rows, 1), lambda i: (i, 0)),
            pl.BlockSpec((block_rows, 1), lambda i: (i, 0)),
        ],
        out_specs=[
            pl.BlockSpec((block_rows, w.shape[1]), lambda i: (i, 0)),
            pl.BlockSpec((block_rows, 1), lambda i: (i, 0)),
        ],
        out_shape=[
            jax.ShapeDtypeStruct((n, w.shape[1]), jnp.float32),
            jax.ShapeDtypeStruct((n, 1), jnp.float32),
        ],
    )(x, w, d0, d1)


def _finish_tc(t0, t1, y, dis, bg, wo, bo, block_rows):
    n, dh = y.shape
    ncls = wo.shape[1]
    grid = n // block_rows

    def body(t0_ref, t1_ref, y_ref, dis_ref, bg_ref, wo_ref, bo_ref,
             h_ref, z_ref):
        t = t0_ref[...] + t1_ref[...] + y_ref[...]
        h = jnp.maximum(t * dis_ref[...] + bg_ref[...], 0.0)
        h_ref[...] = h
        z_ref[...] = jnp.dot(h, wo_ref[...],
                             preferred_element_type=jnp.float32) + bo_ref[...]

    return pl.pallas_call(
        body,
        grid=(grid,),
        in_specs=[
            pl.BlockSpec((block_rows, dh), lambda i: (i, 0)),
            pl.BlockSpec((block_rows, dh), lambda i: (i, 0)),
            pl.BlockSpec((block_rows, dh), lambda i: (i, 0)),
            pl.BlockSpec((block_rows, 1), lambda i: (i, 0)),
            pl.BlockSpec((1, dh), lambda i: (0, 0)),
            pl.BlockSpec((dh, ncls), lambda i: (0, 0)),
            pl.BlockSpec((1, ncls), lambda i: (0, 0)),
        ],
        out_specs=[
            pl.BlockSpec((block_rows, dh), lambda i: (i, 0)),
            pl.BlockSpec((block_rows, ncls), lambda i: (i, 0)),
        ],
        out_shape=[
            jax.ShapeDtypeStruct((n, dh), jnp.float32),
            jax.ShapeDtypeStruct((n, ncls), jnp.float32),
        ],
    )(t0, t1, y, dis, bg, wo, bo)


@jax.jit
def kernel(x, edge_index, W_gcn, b_gcn, W_out, b_out):
    n, _ = x.shape
    dh = W_gcn.shape[1]
    e = edge_index.shape[1]

    n_chunks = -(-e // (NW * LANES))
    e_pad = NW * n_chunks * LANES
    rows_per_tile = -(-(n + 1) // (NS * 8)) * 8
    n_pad = rows_per_tile * NS

    pad = jnp.full((e_pad - e,), n, dtype=jnp.int32)
    src_p = jnp.concatenate([edge_index[0], pad]).reshape(NW, n_chunks, LANES)
    dst_p = jnp.concatenate([edge_index[1], pad]).reshape(NW, n_chunks, LANES)

    ones_h = jnp.ones((LANES, 1), jnp.float32)
    zeros1 = jnp.zeros((n_pad, 1), jnp.float32)
    zeros3 = jnp.zeros((n_pad, dh), jnp.float32)

    deg_parts = _make_deg_kernel(n_chunks, n_pad)(dst_p, ones_h, zeros1)

    y, dis = _y_dis_tc(x, W_gcn, deg_parts[0, :n], deg_parts[1, :n],
                       block_rows=2000)
    y_pad = jnp.concatenate([y, jnp.zeros((n_pad - n, dh), jnp.float32)])

    tmp_parts = _make_scatter_kernel(n_chunks, n_pad, dh)(
        src_p, dst_p, y_pad, zeros3)

    h, z = _finish_tc(tmp_parts[0, :n], tmp_parts[1, :n], y, dis,
                      b_gcn.reshape(1, dh), W_out, b_out.reshape(1, -1),
                      block_rows=2000)
    return (h, z)


# speedup vs baseline: 37.9516x; 37.9516x over previous
import functools

import jax
import jax.numpy as jnp
from jax import lax
from jax.experimental import pallas as pl
from jax.experimental.pallas import tpu as pltpu
from jax.experimental.pallas import tpu_sc as plsc

NC = 2
NS = 16
NW = NC * NS
LANES = 128


def _mesh():
    return plsc.VectorSubcoreMesh(core_axis_name="c", subcore_axis_name="s")


def _make_deg_kernel(n_chunks, n_pad):
    rows_per_tile = n_pad // NS

    @functools.partial(
        pl.kernel,
        out_type=jax.ShapeDtypeStruct((NC, n_pad, 1), jnp.float32),
        mesh=_mesh(),
        compiler_params=pltpu.CompilerParams(use_tc_tiling_on_sc=False),
        scratch_types=[
            pltpu.VMEM((n_chunks, LANES), jnp.int32),
            pltpu.VMEM((LANES, 1), jnp.float32),
            pltpu.VMEM((rows_per_tile, 1), jnp.float32),
            pltpu.VMEM_SHARED((n_pad, 1), jnp.float32),
        ],
    )
    def deg_kernel(dst_hbm, ones_hbm, zeros_hbm, out_hbm, dstv, ones_v, zv, deg_s):
        cid = lax.axis_index("c")
        sid = lax.axis_index("s")
        wid = cid * NS + sid
        r0 = sid * rows_per_tile
        pltpu.sync_copy(dst_hbm.at[wid], dstv)
        pltpu.sync_copy(ones_hbm, ones_v)
        pltpu.sync_copy(zeros_hbm.at[pl.ds(r0, rows_per_tile)], zv)
        pltpu.sync_copy(zv, deg_s.at[pl.ds(r0, rows_per_tile)])
        plsc.subcore_barrier()

        def body(j, carry):
            pltpu.sync_copy(ones_v, deg_s.at[dstv.at[j]], add=True)
            return carry

        lax.fori_loop(0, n_chunks, body, 0)
        plsc.subcore_barrier()
        pltpu.sync_copy(deg_s.at[pl.ds(r0, rows_per_tile)], zv)
        pltpu.sync_copy(zv, out_hbm.at[cid, pl.ds(r0, rows_per_tile)])

    return deg_kernel


def _make_scatter_kernel(n_chunks, n_pad, d):
    rows_per_tile = n_pad // NS

    @functools.partial(
        pl.kernel,
        out_type=jax.ShapeDtypeStruct((NC, n_pad, d), jnp.float32),
        mesh=_mesh(),
        compiler_params=pltpu.CompilerParams(use_tc_tiling_on_sc=False),
        scratch_types=[
            pltpu.VMEM((n_chunks, LANES), jnp.int32),
            pltpu.VMEM((n_chunks, LANES), jnp.int32),
            pltpu.VMEM((LANES, d), jnp.float32),
            pltpu.VMEM((rows_per_tile, d), jnp.float32),
            pltpu.VMEM_SHARED((n_pad, d), jnp.float32),
        ],
    )
    def scatter_kernel(src_hbm, dst_hbm, y_hbm, zeros_hbm, out_hbm,
                       srcv, dstv, ybuf, zv, tmp_s):
        cid = lax.axis_index("c")
        sid = lax.axis_index("s")
        wid = cid * NS + sid
        r0 = sid * rows_per_tile
        pltpu.sync_copy(src_hbm.at[wid], srcv)
        pltpu.sync_copy(dst_hbm.at[wid], dstv)
        pltpu.sync_copy(zeros_hbm.at[pl.ds(r0, rows_per_tile)], zv)
        pltpu.sync_copy(zv, tmp_s.at[pl.ds(r0, rows_per_tile)])
        plsc.subcore_barrier()

        def body(j, carry):
            pltpu.sync_copy(y_hbm.at[srcv.at[j]], ybuf)
            pltpu.sync_copy(ybuf, tmp_s.at[dstv.at[j]], add=True)
            return carry

        lax.fori_loop(0, n_chunks, body, 0)
        plsc.subcore_barrier()
        pltpu.sync_copy(tmp_s.at[pl.ds(r0, rows_per_tile)], zv)
        pltpu.sync_copy(zv, out_hbm.at[cid, pl.ds(r0, rows_per_tile)])

    return scatter_kernel


def _y_dis_tc(x, w, d0, d1, block_rows):
    n = x.shape[0]
    grid = n // block_rows

    def body(x_ref, w_ref, d0_ref, d1_ref, y_ref, dis_ref):
        deg = d0_ref[...] + d1_ref[...] + 1.0
        dis = lax.rsqrt(deg)
        xw = jnp.dot(x_ref[...], w_ref[...], preferred_element_type=jnp.float32)
        y_ref[...] = xw * dis
        dis_ref[...] = dis

    return pl.pallas_call(
        body,
        grid=(grid,),
        in_specs=[
            pl.BlockSpec((block_rows, x.shape[1]), lambda i: (i, 0)),
            pl.BlockSpec((w.shape[0], w.shape[1]), lambda i: (0, 0)),
            pl.BlockSpec((block_rows, 1), lambda i: (i, 0)),
            pl.BlockSpec((block_rows, 1), lambda i: (i, 0)),
        ],
        out_specs=[
            pl.BlockSpec((block_rows, w.shape[1]), lambda i: (i, 0)),
            pl.BlockSpec((block_rows, 1), lambda i: (i, 0)),
        ],
        out_shape=[
            jax.ShapeDtypeStruct((n, w.shape[1]), jnp.float32),
            jax.ShapeDtypeStruct((n, 1), jnp.float32),
        ],
    )(x, w, d0, d1)


def _finish_tc(t0, t1, y, dis, bg, wo, bo, block_rows):
    n, dh = y.shape
    ncls = wo.shape[1]
    grid = n // block_rows

    def body(t0_ref, t1_ref, y_ref, dis_ref, bg_ref, wo_ref, bo_ref,
             h_ref, z_ref):
        t = t0_ref[...] + t1_ref[...] + y_ref[...]
        h = jnp.maximum(t * dis_ref[...] + bg_ref[...], 0.0)
        h_ref[...] = h
        z_ref[...] = jnp.dot(h, wo_ref[...],
                             preferred_element_type=jnp.float32) + bo_ref[...]

    return pl.pallas_call(
        body,
        grid=(grid,),
        in_specs=[
            pl.BlockSpec((block_rows, dh), lambda i: (i, 0)),
            pl.BlockSpec((block_rows, dh), lambda i: (i, 0)),
            pl.BlockSpec((block_rows, dh), lambda i: (i, 0)),
            pl.BlockSpec((block_rows, 1), lambda i: (i, 0)),
            pl.BlockSpec((1, dh), lambda i: (0, 0)),
            pl.BlockSpec((dh, ncls), lambda i: (0, 0)),
            pl.BlockSpec((1, ncls), lambda i: (0, 0)),
        ],
        out_specs=[
            pl.BlockSpec((block_rows, dh), lambda i: (i, 0)),
            pl.BlockSpec((block_rows, ncls), lambda i: (i, 0)),
        ],
        out_shape=[
            jax.ShapeDtypeStruct((n, dh), jnp.float32),
            jax.ShapeDtypeStruct((n, ncls), jnp.float32),
        ],
    )(t0, t1, y, dis, bg, wo, bo)


@jax.jit
def kernel(x, edge_index, W_gcn, b_gcn, W_out, b_out):
    n, _ = x.shape
    dh = W_gcn.shape[1]
    e = edge_index.shape[1]

    n_chunks = -(-e // (NW * LANES))
    e_pad = NW * n_chunks * LANES
    rows_per_tile = -(-(n + 1) // (NS * 8)) * 8
    n_pad = rows_per_tile * NS

    pad = jnp.full((e_pad - e,), n, dtype=jnp.int32)
    src_p = jnp.concatenate([edge_index[0], pad]).reshape(NW, n_chunks, LANES)
    dst_p = jnp.concatenate([edge_index[1], pad]).reshape(NW, n_chunks, LANES)

    ones_h = jnp.ones((LANES, 1), jnp.float32)
    zeros1 = jnp.zeros((n_pad, 1), jnp.float32)
    zeros3 = jnp.zeros((n_pad, dh), jnp.float32)

    deg_parts = _make_deg_kernel(n_chunks, n_pad)(dst_p, ones_h, zeros1)

    y, dis = _y_dis_tc(x, W_gcn, deg_parts[0, :n], deg_parts[1, :n],
                       block_rows=2000)
    y_pad = jnp.concatenate([y, jnp.zeros((n_pad - n, dh), jnp.float32)])

    tmp_parts = _make_scatter_kernel(n_chunks, n_pad, dh)(
        src_p, dst_p, y_pad, zeros3)

    h, z = _finish_tc(tmp_parts[0, :n], tmp_parts[1, :n], y, dis,
                      b_gcn.reshape(1, dh), W_out, b_out.reshape(1, -1),
                      block_rows=2000)
    return (h, z)


# speedup vs baseline: 51.6486x; 1.3609x over previous
import functools

import jax
import jax.numpy as jnp
from jax import lax
from jax.experimental import pallas as pl
from jax.experimental.pallas import tpu as pltpu
from jax.experimental.pallas import tpu_sc as plsc

NC = 2
NS = 16
NW = NC * NS
LANES = 128


def _mesh():
    return plsc.VectorSubcoreMesh(core_axis_name="c", subcore_axis_name="s")


def _make_deg_kernel(n_chunks, n_pad):
    rows_per_tile = n_pad // NS

    @functools.partial(
        pl.kernel,
        out_type=jax.ShapeDtypeStruct((NC, n_pad, 1), jnp.float32),
        mesh=_mesh(),
        compiler_params=pltpu.CompilerParams(use_tc_tiling_on_sc=False),
        scratch_types=[
            pltpu.VMEM((n_chunks, LANES), jnp.int32),
            pltpu.VMEM((LANES, 1), jnp.float32),
            pltpu.VMEM((rows_per_tile, 1), jnp.float32),
            pltpu.VMEM_SHARED((n_pad, 1), jnp.float32),
        ],
    )
    def deg_kernel(dst_hbm, ones_hbm, zeros_hbm, out_hbm, dstv, ones_v, zv, deg_s):
        cid = lax.axis_index("c")
        sid = lax.axis_index("s")
        wid = cid * NS + sid
        r0 = sid * rows_per_tile
        pltpu.sync_copy(dst_hbm.at[wid], dstv)
        pltpu.sync_copy(ones_hbm, ones_v)
        pltpu.sync_copy(zeros_hbm.at[pl.ds(r0, rows_per_tile)], zv)
        pltpu.sync_copy(zv, deg_s.at[pl.ds(r0, rows_per_tile)])
        plsc.subcore_barrier()

        def body(j, carry):
            pltpu.sync_copy(ones_v, deg_s.at[dstv.at[j]], add=True)
            return carry

        lax.fori_loop(0, n_chunks, body, 0)
        plsc.subcore_barrier()
        pltpu.sync_copy(deg_s.at[pl.ds(r0, rows_per_tile)], zv)
        pltpu.sync_copy(zv, out_hbm.at[cid, pl.ds(r0, rows_per_tile)])

    return deg_kernel


def _make_scatter_kernel(n_chunks, n_pad, d):
    rows_per_tile = n_pad // NS

    @functools.partial(
        pl.kernel,
        out_type=jax.ShapeDtypeStruct((NC, n_pad, d), jnp.float32),
        mesh=_mesh(),
        compiler_params=pltpu.CompilerParams(use_tc_tiling_on_sc=False),
        scratch_types=[
            pltpu.VMEM((n_chunks, LANES), jnp.int32),
            pltpu.VMEM((n_chunks, LANES), jnp.int32),
            pltpu.VMEM((LANES, d), jnp.float32),
            pltpu.VMEM((rows_per_tile, d), jnp.float32),
            pltpu.VMEM_SHARED((n_pad, d), jnp.float32),
            pltpu.VMEM_SHARED((n_pad, d), jnp.float32),
        ],
    )
    def scatter_kernel(src_hbm, dst_hbm, y_hbm, zeros_hbm, out_hbm,
                       srcv, dstv, ybuf, zv, tmp_s, y_s):
        cid = lax.axis_index("c")
        sid = lax.axis_index("s")
        wid = cid * NS + sid
        r0 = sid * rows_per_tile
        pltpu.sync_copy(src_hbm.at[wid], srcv)
        pltpu.sync_copy(dst_hbm.at[wid], dstv)
        pltpu.sync_copy(zeros_hbm.at[pl.ds(r0, rows_per_tile)], zv)
        pltpu.sync_copy(zv, tmp_s.at[pl.ds(r0, rows_per_tile)])
        pltpu.sync_copy(y_hbm.at[pl.ds(r0, rows_per_tile)], zv)
        pltpu.sync_copy(zv, y_s.at[pl.ds(r0, rows_per_tile)])
        plsc.subcore_barrier()

        def body(j, carry):
            pltpu.sync_copy(y_s.at[srcv.at[j]], ybuf)
            pltpu.sync_copy(ybuf, tmp_s.at[dstv.at[j]], add=True)
            return carry

        lax.fori_loop(0, n_chunks, body, 0)
        plsc.subcore_barrier()
        pltpu.sync_copy(tmp_s.at[pl.ds(r0, rows_per_tile)], zv)
        pltpu.sync_copy(zv, out_hbm.at[cid, pl.ds(r0, rows_per_tile)])

    return scatter_kernel


def _y_dis_tc(x, w, d0, d1, block_rows):
    n = x.shape[0]
    grid = n // block_rows

    def body(x_ref, w_ref, d0_ref, d1_ref, y_ref, dis_ref):
        deg = d0_ref[...] + d1_ref[...] + 1.0
        dis = lax.rsqrt(deg)
        xw = jnp.dot(x_ref[...], w_ref[...], preferred_element_type=jnp.float32)
        y_ref[...] = xw * dis
        dis_ref[...] = dis

    return pl.pallas_call(
        body,
        grid=(grid,),
        in_specs=[
            pl.BlockSpec((block_rows, x.shape[1]), lambda i: (i, 0)),
            pl.BlockSpec((w.shape[0], w.shape[1]), lambda i: (0, 0)),
            pl.BlockSpec((block_rows, 1), lambda i: (i, 0)),
            pl.BlockSpec((block_rows, 1), lambda i: (i, 0)),
        ],
        out_specs=[
            pl.BlockSpec((block_rows, w.shape[1]), lambda i: (i, 0)),
            pl.BlockSpec((block_rows, 1), lambda i: (i, 0)),
        ],
        out_shape=[
            jax.ShapeDtypeStruct((n, w.shape[1]), jnp.float32),
            jax.ShapeDtypeStruct((n, 1), jnp.float32),
        ],
    )(x, w, d0, d1)


def _finish_tc(t0, t1, y, dis, bg, wo, bo, block_rows):
    n, dh = y.shape
    ncls = wo.shape[1]
    grid = n // block_rows

    def body(t0_ref, t1_ref, y_ref, dis_ref, bg_ref, wo_ref, bo_ref,
             h_ref, z_ref):
        t = t0_ref[...] + t1_ref[...] + y_ref[...]
        h = jnp.maximum(t * dis_ref[...] + bg_ref[...], 0.0)
        h_ref[...] = h
        z_ref[...] = jnp.dot(h, wo_ref[...],
                             preferred_element_type=jnp.float32) + bo_ref[...]

    return pl.pallas_call(
        body,
        grid=(grid,),
        in_specs=[
            pl.BlockSpec((block_rows, dh), lambda i: (i, 0)),
            pl.BlockSpec((block_rows, dh), lambda i: (i, 0)),
            pl.BlockSpec((block_rows, dh), lambda i: (i, 0)),
            pl.BlockSpec((block_rows, 1), lambda i: (i, 0)),
            pl.BlockSpec((1, dh), lambda i: (0, 0)),
            pl.BlockSpec((dh, ncls), lambda i: (0, 0)),
            pl.BlockSpec((1, ncls), lambda i: (0, 0)),
        ],
        out_specs=[
            pl.BlockSpec((block_rows, dh), lambda i: (i, 0)),
            pl.BlockSpec((block_rows, ncls), lambda i: (i, 0)),
        ],
        out_shape=[
            jax.ShapeDtypeStruct((n, dh), jnp.float32),
            jax.ShapeDtypeStruct((n, ncls), jnp.float32),
        ],
    )(t0, t1, y, dis, bg, wo, bo)


@jax.jit
def kernel(x, edge_index, W_gcn, b_gcn, W_out, b_out):
    n, _ = x.shape
    dh = W_gcn.shape[1]
    e = edge_index.shape[1]

    n_chunks = -(-e // (NW * LANES))
    e_pad = NW * n_chunks * LANES
    rows_per_tile = -(-(n + 1) // (NS * 8)) * 8
    n_pad = rows_per_tile * NS

    pad = jnp.full((e_pad - e,), n, dtype=jnp.int32)
    src_p = jnp.concatenate([edge_index[0], pad]).reshape(NW, n_chunks, LANES)
    dst_p = jnp.concatenate([edge_index[1], pad]).reshape(NW, n_chunks, LANES)

    ones_h = jnp.ones((LANES, 1), jnp.float32)
    zeros1 = jnp.zeros((n_pad, 1), jnp.float32)
    zeros3 = jnp.zeros((n_pad, dh), jnp.float32)

    deg_parts = _make_deg_kernel(n_chunks, n_pad)(dst_p, ones_h, zeros1)

    y, dis = _y_dis_tc(x, W_gcn, deg_parts[0, :n], deg_parts[1, :n],
                       block_rows=2000)
    y_pad = jnp.concatenate([y, jnp.zeros((n_pad - n, dh), jnp.float32)])

    tmp_parts = _make_scatter_kernel(n_chunks, n_pad, dh)(
        src_p, dst_p, y_pad, zeros3)

    h, z = _finish_tc(tmp_parts[0, :n], tmp_parts[1, :n], y, dis,
                      b_gcn.reshape(1, dh), W_out, b_out.reshape(1, -1),
                      block_rows=2000)
    return (h, z)


# speedup vs baseline: 54.5438x; 1.0561x over previous
import functools

import jax
import jax.numpy as jnp
from jax import lax
from jax.experimental import pallas as pl
from jax.experimental.pallas import tpu as pltpu
from jax.experimental.pallas import tpu_sc as plsc

NC = 2
NS = 16
NW = NC * NS
LANES = 128


def _mesh():
    return plsc.VectorSubcoreMesh(core_axis_name="c", subcore_axis_name="s")


def _make_deg_kernel(n_chunks, n_pad):
    rows_per_tile = n_pad // NS

    @functools.partial(
        pl.kernel,
        out_type=jax.ShapeDtypeStruct((NC, n_pad, 1), jnp.float32),
        mesh=_mesh(),
        compiler_params=pltpu.CompilerParams(use_tc_tiling_on_sc=False),
        scratch_types=[
            pltpu.VMEM((n_chunks, LANES), jnp.int32),
            pltpu.VMEM((LANES, 1), jnp.float32),
            pltpu.VMEM((rows_per_tile, 1), jnp.float32),
            pltpu.VMEM_SHARED((n_pad, 1), jnp.float32),
        ],
    )
    def deg_kernel(dst_hbm, ones_hbm, zeros_hbm, out_hbm, dstv, ones_v, zv, deg_s):
        cid = lax.axis_index("c")
        sid = lax.axis_index("s")
        wid = cid * NS + sid
        r0 = sid * rows_per_tile
        pltpu.sync_copy(dst_hbm.at[wid], dstv)
        pltpu.sync_copy(ones_hbm, ones_v)
        pltpu.sync_copy(zeros_hbm.at[pl.ds(r0, rows_per_tile)], zv)
        pltpu.sync_copy(zv, deg_s.at[pl.ds(r0, rows_per_tile)])
        plsc.subcore_barrier()

        def body(j, carry):
            pltpu.sync_copy(ones_v, deg_s.at[dstv.at[j]], add=True)
            return carry

        lax.fori_loop(0, n_chunks, body, 0)
        plsc.subcore_barrier()
        pltpu.sync_copy(deg_s.at[pl.ds(r0, rows_per_tile)], zv)
        pltpu.sync_copy(zv, out_hbm.at[cid, pl.ds(r0, rows_per_tile)])

    return deg_kernel


def _make_scatter_kernel(n_chunks, n_pad, d):
    rows_per_tile = n_pad // NS

    @functools.partial(
        pl.kernel,
        out_type=jax.ShapeDtypeStruct((NC, n_pad, d), jnp.float32),
        mesh=_mesh(),
        compiler_params=pltpu.CompilerParams(use_tc_tiling_on_sc=False),
        scratch_types=[
            pltpu.VMEM((n_chunks, LANES), jnp.int32),
            pltpu.VMEM((n_chunks, LANES), jnp.int32),
            pltpu.VMEM((LANES, d), jnp.float32),
            pltpu.VMEM((LANES, d), jnp.float32),
            pltpu.VMEM((rows_per_tile, d), jnp.float32),
            pltpu.VMEM_SHARED((n_pad, d), jnp.float32),
            pltpu.VMEM_SHARED((n_pad, d), jnp.float32),
            pltpu.SemaphoreType.DMA,
            pltpu.SemaphoreType.DMA,
        ],
    )
    def scatter_kernel(src_hbm, dst_hbm, y_hbm, zeros_hbm, out_hbm,
                       srcv, dstv, ybuf0, ybuf1, zv, tmp_s, y_s, sem0, sem1):
        cid = lax.axis_index("c")
        sid = lax.axis_index("s")
        wid = cid * NS + sid
        r0 = sid * rows_per_tile
        pltpu.sync_copy(src_hbm.at[wid], srcv)
        pltpu.sync_copy(dst_hbm.at[wid], dstv)
        pltpu.sync_copy(zeros_hbm.at[pl.ds(r0, rows_per_tile)], zv)
        pltpu.sync_copy(zv, tmp_s.at[pl.ds(r0, rows_per_tile)])
        pltpu.sync_copy(y_hbm.at[pl.ds(r0, rows_per_tile)], zv)
        pltpu.sync_copy(zv, y_s.at[pl.ds(r0, rows_per_tile)])
        plsc.subcore_barrier()

        pltpu.async_copy(y_s.at[srcv.at[0]], ybuf0, sem0)

        def body(j2, carry):
            j = j2 * 2
            pltpu.make_async_copy(y_s.at[srcv.at[j]], ybuf0, sem0).wait()
            pltpu.async_copy(y_s.at[srcv.at[j + 1]], ybuf1, sem1)
            pltpu.sync_copy(ybuf0, tmp_s.at[dstv.at[j]], add=True)
            pltpu.make_async_copy(y_s.at[srcv.at[j + 1]], ybuf1, sem1).wait()
            pltpu.async_copy(y_s.at[srcv.at[j + 2]], ybuf0, sem0)
            pltpu.sync_copy(ybuf1, tmp_s.at[dstv.at[j + 1]], add=True)
            return carry

        lax.fori_loop(0, (n_chunks - 1) // 2, body, 0)
        j_last = n_chunks - 1
        pltpu.make_async_copy(y_s.at[srcv.at[j_last]], ybuf0, sem0).wait()
        pltpu.sync_copy(ybuf0, tmp_s.at[dstv.at[j_last]], add=True)
        plsc.subcore_barrier()
        pltpu.sync_copy(tmp_s.at[pl.ds(r0, rows_per_tile)], zv)
        pltpu.sync_copy(zv, out_hbm.at[cid, pl.ds(r0, rows_per_tile)])

    return scatter_kernel


def _y_dis_tc(x, w, d0, d1, block_rows):
    n = x.shape[0]
    grid = n // block_rows

    def body(x_ref, w_ref, d0_ref, d1_ref, y_ref, dis_ref):
        deg = d0_ref[...] + d1_ref[...] + 1.0
        dis = lax.rsqrt(deg)
        xw = jnp.dot(x_ref[...], w_ref[...], preferred_element_type=jnp.float32)
        y_ref[...] = xw * dis
        dis_ref[...] = dis

    return pl.pallas_call(
        body,
        grid=(grid,),
        in_specs=[
            pl.BlockSpec((block_rows, x.shape[1]), lambda i: (i, 0)),
            pl.BlockSpec((w.shape[0], w.shape[1]), lambda i: (0, 0)),
            pl.BlockSpec((block_rows, 1), lambda i: (i, 0)),
            pl.BlockSpec((block_rows, 1), lambda i: (i, 0)),
        ],
        out_specs=[
            pl.BlockSpec((block_rows, w.shape[1]), lambda i: (i, 0)),
            pl.BlockSpec((block_rows, 1), lambda i: (i, 0)),
        ],
        out_shape=[
            jax.ShapeDtypeStruct((n, w.shape[1]), jnp.float32),
            jax.ShapeDtypeStruct((n, 1), jnp.float32),
        ],
    )(x, w, d0, d1)


def _finish_tc(t0, t1, y, dis, bg, wo, bo, block_rows):
    n, dh = y.shape
    ncls = wo.shape[1]
    grid = n // block_rows

    def body(t0_ref, t1_ref, y_ref, dis_ref, bg_ref, wo_ref, bo_ref,
             h_ref, z_ref):
        t = t0_ref[...] + t1_ref[...] + y_ref[...]
        h = jnp.maximum(t * dis_ref[...] + bg_ref[...], 0.0)
        h_ref[...] = h
        z_ref[...] = jnp.dot(h, wo_ref[...],
                             preferred_element_type=jnp.float32) + bo_ref[...]

    return pl.pallas_call(
        body,
        grid=(grid,),
        in_specs=[
            pl.BlockSpec((block_rows, dh), lambda i: (i, 0)),
            pl.BlockSpec((block_rows, dh), lambda i: (i, 0)),
            pl.BlockSpec((block_rows, dh), lambda i: (i, 0)),
            pl.BlockSpec((block_rows, 1), lambda i: (i, 0)),
            pl.BlockSpec((1, dh), lambda i: (0, 0)),
            pl.BlockSpec((dh, ncls), lambda i: (0, 0)),
            pl.BlockSpec((1, ncls), lambda i: (0, 0)),
        ],
        out_specs=[
            pl.BlockSpec((block_rows, dh), lambda i: (i, 0)),
            pl.BlockSpec((block_rows, ncls), lambda i: (i, 0)),
        ],
        out_shape=[
            jax.ShapeDtypeStruct((n, dh), jnp.float32),
            jax.ShapeDtypeStruct((n, ncls), jnp.float32),
        ],
    )(t0, t1, y, dis, bg, wo, bo)


@jax.jit
def kernel(x, edge_index, W_gcn, b_gcn, W_out, b_out):
    n, _ = x.shape
    dh = W_gcn.shape[1]
    e = edge_index.shape[1]

    n_chunks = -(-e // (NW * LANES))
    if n_chunks % 2 == 0:
        n_chunks += 1
    e_pad = NW * n_chunks * LANES
    rows_per_tile = -(-(n + 1) // (NS * 8)) * 8
    n_pad = rows_per_tile * NS

    pad = jnp.full((e_pad - e,), n, dtype=jnp.int32)
    src_p = jnp.concatenate([edge_index[0], pad]).reshape(NW, n_chunks, LANES)
    dst_p = jnp.concatenate([edge_index[1], pad]).reshape(NW, n_chunks, LANES)

    ones_h = jnp.ones((LANES, 1), jnp.float32)
    zeros1 = jnp.zeros((n_pad, 1), jnp.float32)
    zeros3 = jnp.zeros((n_pad, dh), jnp.float32)

    deg_parts = _make_deg_kernel(n_chunks, n_pad)(dst_p, ones_h, zeros1)

    y, dis = _y_dis_tc(x, W_gcn, deg_parts[0, :n], deg_parts[1, :n],
                       block_rows=2000)
    y_pad = jnp.concatenate([y, jnp.zeros((n_pad - n, dh), jnp.float32)])

    tmp_parts = _make_scatter_kernel(n_chunks, n_pad, dh)(
        src_p, dst_p, y_pad, zeros3)

    h, z = _finish_tc(tmp_parts[0, :n], tmp_parts[1, :n], y, dis,
                      b_gcn.reshape(1, dh), W_out, b_out.reshape(1, -1),
                      block_rows=2000)
    return (h, z)


# speedup vs baseline: 58.9915x; 1.0815x over previous
import functools

import jax
import jax.numpy as jnp
from jax import lax
from jax.experimental import pallas as pl
from jax.experimental.pallas import tpu as pltpu
from jax.experimental.pallas import tpu_sc as plsc

NC = 2
NS = 16
NW = NC * NS
LANES = 128


def _mesh():
    return plsc.VectorSubcoreMesh(core_axis_name="c", subcore_axis_name="s")


def _make_deg_kernel(n_chunks, n_pad):
    rows_per_tile = n_pad // NS

    @functools.partial(
        pl.kernel,
        out_type=jax.ShapeDtypeStruct((NC, n_pad, 1), jnp.float32),
        mesh=_mesh(),
        compiler_params=pltpu.CompilerParams(use_tc_tiling_on_sc=False),
        scratch_types=[
            pltpu.VMEM((n_chunks, LANES), jnp.int32),
            pltpu.VMEM((LANES, 1), jnp.float32),
            pltpu.VMEM((rows_per_tile, 1), jnp.float32),
            pltpu.VMEM_SHARED((n_pad, 1), jnp.float32),
            pltpu.SemaphoreType.DMA,
            pltpu.SemaphoreType.DMA,
        ],
    )
    def deg_kernel(dst_hbm, ones_hbm, zeros_hbm, out_hbm, dstv, ones_v, zv,
                   deg_s, sem0, sem1):
        cid = lax.axis_index("c")
        sid = lax.axis_index("s")
        wid = cid * NS + sid
        r0 = sid * rows_per_tile
        pltpu.sync_copy(dst_hbm.at[wid], dstv)
        pltpu.sync_copy(ones_hbm, ones_v)
        pltpu.sync_copy(zeros_hbm.at[pl.ds(r0, rows_per_tile)], zv)
        pltpu.sync_copy(zv, deg_s.at[pl.ds(r0, rows_per_tile)])
        plsc.subcore_barrier()

        pltpu.async_copy(ones_v, deg_s.at[dstv.at[0]], sem0, add=True)

        def body(j2, carry):
            j = j2 * 2
            pltpu.async_copy(ones_v, deg_s.at[dstv.at[j + 1]], sem1, add=True)
            pltpu.make_async_copy(ones_v, deg_s.at[dstv.at[j]], sem0).wait()
            pltpu.async_copy(ones_v, deg_s.at[dstv.at[j + 2]], sem0, add=True)
            pltpu.make_async_copy(ones_v, deg_s.at[dstv.at[j + 1]], sem1).wait()
            return carry

        lax.fori_loop(0, (n_chunks - 1) // 2, body, 0)
        pltpu.make_async_copy(ones_v, deg_s.at[dstv.at[n_chunks - 1]],
                              sem0).wait()
        plsc.subcore_barrier()
        pltpu.sync_copy(deg_s.at[pl.ds(r0, rows_per_tile)], zv)
        pltpu.sync_copy(zv, out_hbm.at[cid, pl.ds(r0, rows_per_tile)])

    return deg_kernel


def _make_scatter_kernel(n_chunks, n_pad, d):
    rows_per_tile = n_pad // NS

    @functools.partial(
        pl.kernel,
        out_type=jax.ShapeDtypeStruct((NC, n_pad, d), jnp.float32),
        mesh=_mesh(),
        compiler_params=pltpu.CompilerParams(use_tc_tiling_on_sc=False),
        scratch_types=[
            pltpu.VMEM((n_chunks, LANES), jnp.int32),
            pltpu.VMEM((n_chunks, LANES), jnp.int32),
            pltpu.VMEM((LANES, d), jnp.float32),
            pltpu.VMEM((LANES, d), jnp.float32),
            pltpu.VMEM((rows_per_tile, d), jnp.float32),
            pltpu.VMEM_SHARED((n_pad, d), jnp.float32),
            pltpu.VMEM_SHARED((n_pad, d), jnp.float32),
            pltpu.SemaphoreType.DMA,
            pltpu.SemaphoreType.DMA,
        ],
    )
    def scatter_kernel(src_hbm, dst_hbm, y_hbm, zeros_hbm, out_hbm,
                       srcv, dstv, ybuf0, ybuf1, zv, tmp_s, y_s, sem0, sem1):
        cid = lax.axis_index("c")
        sid = lax.axis_index("s")
        wid = cid * NS + sid
        r0 = sid * rows_per_tile
        pltpu.sync_copy(src_hbm.at[wid], srcv)
        pltpu.sync_copy(dst_hbm.at[wid], dstv)
        pltpu.sync_copy(zeros_hbm.at[pl.ds(r0, rows_per_tile)], zv)
        pltpu.sync_copy(zv, tmp_s.at[pl.ds(r0, rows_per_tile)])
        pltpu.sync_copy(y_hbm.at[pl.ds(r0, rows_per_tile)], zv)
        pltpu.sync_copy(zv, y_s.at[pl.ds(r0, rows_per_tile)])
        plsc.subcore_barrier()

        pltpu.async_copy(y_s.at[srcv.at[0]], ybuf0, sem0)

        def body(j2, carry):
            j = j2 * 2
            pltpu.make_async_copy(y_s.at[srcv.at[j]], ybuf0, sem0).wait()
            pltpu.async_copy(y_s.at[srcv.at[j + 1]], ybuf1, sem1)
            pltpu.sync_copy(ybuf0, tmp_s.at[dstv.at[j]], add=True)
            pltpu.make_async_copy(y_s.at[srcv.at[j + 1]], ybuf1, sem1).wait()
            pltpu.async_copy(y_s.at[srcv.at[j + 2]], ybuf0, sem0)
            pltpu.sync_copy(ybuf1, tmp_s.at[dstv.at[j + 1]], add=True)
            return carry

        lax.fori_loop(0, (n_chunks - 1) // 2, body, 0)
        j_last = n_chunks - 1
        pltpu.make_async_copy(y_s.at[srcv.at[j_last]], ybuf0, sem0).wait()
        pltpu.sync_copy(ybuf0, tmp_s.at[dstv.at[j_last]], add=True)
        plsc.subcore_barrier()
        pltpu.sync_copy(tmp_s.at[pl.ds(r0, rows_per_tile)], zv)
        pltpu.sync_copy(zv, out_hbm.at[cid, pl.ds(r0, rows_per_tile)])

    return scatter_kernel


def _y_dis_tc(x, w, deg_parts, n_pad, block_rows):
    grid = n_pad // block_rows

    def body(x_ref, w_ref, d0_ref, d1_ref, y_ref, dis_ref):
        deg = d0_ref[0] + d1_ref[0] + 1.0
        dis = lax.rsqrt(deg)
        xw = jnp.dot(x_ref[...], w_ref[...], preferred_element_type=jnp.float32)
        y_ref[...] = xw * dis
        dis_ref[...] = dis

    return pl.pallas_call(
        body,
        grid=(grid,),
        in_specs=[
            pl.BlockSpec((block_rows, x.shape[1]), lambda i: (i, 0)),
            pl.BlockSpec((w.shape[0], w.shape[1]), lambda i: (0, 0)),
            pl.BlockSpec((1, block_rows, 1), lambda i: (0, i, 0)),
            pl.BlockSpec((1, block_rows, 1), lambda i: (1, i, 0)),
        ],
        out_specs=[
            pl.BlockSpec((block_rows, w.shape[1]), lambda i: (i, 0)),
            pl.BlockSpec((block_rows, 1), lambda i: (i, 0)),
        ],
        out_shape=[
            jax.ShapeDtypeStruct((n_pad, w.shape[1]), jnp.float32),
            jax.ShapeDtypeStruct((n_pad, 1), jnp.float32),
        ],
    )(x, w, deg_parts, deg_parts)


def _finish_tc(tmp_parts, y, dis, bg, wo, bo, n, block_rows):
    dh = y.shape[1]
    ncls = wo.shape[1]
    grid = n // block_rows

    def body(t0_ref, t1_ref, y_ref, dis_ref, bg_ref, wo_ref, bo_ref,
             h_ref, z_ref):
        t = t0_ref[0] + t1_ref[0] + y_ref[...]
        h = jnp.maximum(t * dis_ref[...] + bg_ref[...], 0.0)
        h_ref[...] = h
        z_ref[...] = jnp.dot(h, wo_ref[...],
                             preferred_element_type=jnp.float32) + bo_ref[...]

    return pl.pallas_call(
        body,
        grid=(grid,),
        in_specs=[
            pl.BlockSpec((1, block_rows, dh), lambda i: (0, i, 0)),
            pl.BlockSpec((1, block_rows, dh), lambda i: (1, i, 0)),
            pl.BlockSpec((block_rows, dh), lambda i: (i, 0)),
            pl.BlockSpec((block_rows, 1), lambda i: (i, 0)),
            pl.BlockSpec((1, dh), lambda i: (0, 0)),
            pl.BlockSpec((dh, ncls), lambda i: (0, 0)),
            pl.BlockSpec((1, ncls), lambda i: (0, 0)),
        ],
        out_specs=[
            pl.BlockSpec((block_rows, dh), lambda i: (i, 0)),
            pl.BlockSpec((block_rows, ncls), lambda i: (i, 0)),
        ],
        out_shape=[
            jax.ShapeDtypeStruct((n, dh), jnp.float32),
            jax.ShapeDtypeStruct((n, ncls), jnp.float32),
        ],
    )(tmp_parts, tmp_parts, y, dis, bg, wo, bo)


@jax.jit
def kernel(x, edge_index, W_gcn, b_gcn, W_out, b_out):
    n, _ = x.shape
    dh = W_gcn.shape[1]
    e = edge_index.shape[1]

    n_chunks = -(-e // (NW * LANES))
    if n_chunks % 2 == 0:
        n_chunks += 1
    e_pad = NW * n_chunks * LANES
    rows_per_tile = -(-(n + 1) // (NS * 8)) * 8
    n_pad = rows_per_tile * NS

    pad = jnp.full((e_pad - e,), n, dtype=jnp.int32)
    src_p = jnp.concatenate([edge_index[0], pad]).reshape(NW, n_chunks, LANES)
    dst_p = jnp.concatenate([edge_index[1], pad]).reshape(NW, n_chunks, LANES)

    ones_h = jnp.ones((LANES, 1), jnp.float32)
    zeros1 = jnp.zeros((n_pad, 1), jnp.float32)
    zeros3 = jnp.zeros((n_pad, dh), jnp.float32)

    deg_parts = _make_deg_kernel(n_chunks, n_pad)(dst_p, ones_h, zeros1)

    y, dis = _y_dis_tc(x, W_gcn, deg_parts, n_pad, block_rows=n_pad // 16)

    tmp_parts = _make_scatter_kernel(n_chunks, n_pad, dh)(
        src_p, dst_p, y, zeros3)

    h, z = _finish_tc(tmp_parts, y, dis,
                      b_gcn.reshape(1, dh), W_out, b_out.reshape(1, -1),
                      n, block_rows=2000)
    return (h, z)
